# Initial kernel scaffold; baseline (speedup 1.0000x reference)
import jax
import jax.numpy as jnp
from jax.experimental import pallas as pl


def kernel(X, W, b):
    raise NotImplementedError("write your pallas kernel here")


# speedup vs baseline: 18.8312x; 18.8312x over previous
import functools

import jax
import jax.numpy as jnp
from jax.experimental import pallas as pl

_EPS = 1e-06
_CAPACITY_FACTOR = 1.0


def _gate_kernel(x_ref, w_ref, b_ref, o_ref, *, capacity):
    bsz, cs, dim = x_ref.shape
    ne = w_ref.shape[1]
    x2 = x_ref[...].reshape(bsz * cs, dim)
    logits = jnp.dot(x2, w_ref[...], preferred_element_type=jnp.float32)
    logits = logits + b_ref[...]
    m = jnp.max(logits, axis=-1, keepdims=True)
    e = jnp.exp(logits - m)
    g = e / jnp.sum(e, axis=-1, keepdims=True)
    g3 = g.reshape(bsz, cs, ne)
    den = jnp.sum(g3, axis=0, keepdims=True) + _EPS
    o_ref[...] = g3 * (capacity / den)


@functools.partial(jax.jit, static_argnames=())
def kernel(X, W, b):
    bsz, seq, dim = X.shape
    ne = W.shape[1]
    capacity = float(int(_CAPACITY_FACTOR * bsz))
    cs = 256
    b2 = b.reshape(1, ne)
    grid = (seq // cs,)
    return pl.pallas_call(
        functools.partial(_gate_kernel, capacity=capacity),
        grid=grid,
        in_specs=[
            pl.BlockSpec((bsz, cs, dim), lambda i: (0, i, 0)),
            pl.BlockSpec((dim, ne), lambda i: (0, 0)),
            pl.BlockSpec((1, ne), lambda i: (0, 0)),
        ],
        out_specs=pl.BlockSpec((bsz, cs, ne), lambda i: (0, i, 0)),
        out_shape=jax.ShapeDtypeStruct((bsz, seq, ne), jnp.float32),
    )(X, W, b2)
